# Initial kernel scaffold; baseline (speedup 1.0000x reference)
import functools

import jax
import jax.numpy as jnp
from jax import lax
from jax.experimental import pallas as pl
from jax.experimental.pallas import tpu as pltpu
from jax.experimental.pallas import tpu_sc as plsc

N = 10000
E = 160000
DF = 256
DE = 16
DG = 128
G = 16
H = 512
EO = 256
NO = 256
GO = 128

f32 = jnp.float32
i32 = jnp.int32

NC = 2
NS = 16
L = 16
NW = NC * NS


_UV_TILE = 2000


def _uv_body(x_ref, ws_ref, wd_ref, u_ref, v_ref):
    xv = x_ref[...]
    u_ref[...] = jnp.dot(xv, ws_ref[...], preferred_element_type=f32)
    v_ref[...] = jnp.dot(xv, wd_ref[...], preferred_element_type=f32)


def _compute_uv(x, Ws, Wd):
    nb = N // _UV_TILE
    return pl.pallas_call(
        _uv_body,
        grid=(nb,),
        in_specs=[
            pl.BlockSpec((_UV_TILE, DF), lambda i: (i, 0)),
            pl.BlockSpec((DF, H), lambda i: (0, 0)),
            pl.BlockSpec((DF, H), lambda i: (0, 0)),
        ],
        out_specs=[
            pl.BlockSpec((_UV_TILE, H), lambda i: (i, 0)),
            pl.BlockSpec((_UV_TILE, H), lambda i: (i, 0)),
        ],
        out_shape=[
            jax.ShapeDtypeStruct((N, H), f32),
            jax.ShapeDtypeStruct((N, H), f32),
        ],
    )(x, Ws, Wd)


_KG = 40
_EPW = E // NW


def _gather_h1p(U, V, src, dst):
    mesh = plsc.VectorSubcoreMesh(core_axis_name="c", subcore_axis_name="s")

    @functools.partial(
        pl.kernel,
        mesh=mesh,
        out_type=jax.ShapeDtypeStruct((E, H), f32),
        scratch_types=[
            pltpu.VMEM((_EPW,), i32),
            pltpu.VMEM((_EPW,), i32),
            pltpu.VMEM((_KG, H), f32),
            pltpu.VMEM((_KG, H), f32),
            pltpu.SemaphoreType.DMA,
            pltpu.SemaphoreType.DMA,
        ],
    )
    def k(u_hbm, v_hbm, src_hbm, dst_hbm, out_hbm, si, di, bu, bv, sem_u, sem_v):
        wid = lax.axis_index("s") * NC + lax.axis_index("c")
        base = wid * _EPW
        pltpu.sync_copy(src_hbm.at[pl.ds(base, _EPW)], si)
        pltpu.sync_copy(dst_hbm.at[pl.ds(base, _EPW)], di)

        def chunk(j, carry):
            off = j * _KG
            cu = pltpu.async_copy(u_hbm.at[si.at[pl.ds(off, _KG)]], bu, sem_u)
            cv = pltpu.async_copy(v_hbm.at[di.at[pl.ds(off, _KG)]], bv, sem_v)
            cu.wait()
            cv.wait()

            def addrow(r, c2):
                for q in range(H // L):
                    sl = pl.ds(q * L, L)
                    bu[r, sl] = bu[r, sl] + bv[r, sl]
                return c2

            lax.fori_loop(0, _KG, addrow, 0)
            pltpu.sync_copy(bu, out_hbm.at[pl.ds(base + off, _KG)])
            return carry

        lax.fori_loop(0, _EPW // _KG, chunk, 0)

    return k(U, V, src, dst)


_TE = 640


def _edge_body(h_ref, e_ref, w1_ref, b1_ref, w2_ref, b2_ref, idx_ref,
               ea_ref, eg_ref):
    i = pl.program_id(0)
    h1 = (h_ref[...]
          + jnp.dot(e_ref[...], w1_ref[...], preferred_element_type=f32)
          + b1_ref[...])
    a = jnp.maximum(h1, 0.0)
    ea = jnp.dot(a, w2_ref[...], preferred_element_type=f32) + b2_ref[...]
    ea_ref[...] = ea
    gcol = lax.broadcasted_iota(i32, (G, _TE), 0)
    oh = (gcol == idx_ref[0]).astype(f32)
    part = jnp.dot(oh, ea, preferred_element_type=f32)

    @pl.when(i == 0)
    def _():
        eg_ref[...] = part

    @pl.when(i != 0)
    def _():
        eg_ref[...] = eg_ref[...] + part


def _edge_block(h1p, e, We1e, be1, We2, be2, edge_idx):
    nb = E // _TE
    idx3 = edge_idx.reshape(nb, 1, _TE)
    return pl.pallas_call(
        _edge_body,
        grid=(nb,),
        in_specs=[
            pl.BlockSpec((_TE, H), lambda i: (i, 0)),
            pl.BlockSpec((_TE, DE), lambda i: (i, 0)),
            pl.BlockSpec((DE, H), lambda i: (0, 0)),
            pl.BlockSpec((1, H), lambda i: (0, 0)),
            pl.BlockSpec((H, EO), lambda i: (0, 0)),
            pl.BlockSpec((1, EO), lambda i: (0, 0)),
            pl.BlockSpec((1, 1, _TE), lambda i: (i, 0, 0)),
        ],
        out_specs=[
            pl.BlockSpec((_TE, EO), lambda i: (i, 0)),
            pl.BlockSpec((G, EO), lambda i: (0, 0)),
        ],
        out_shape=[
            jax.ShapeDtypeStruct((E, EO), f32),
            jax.ShapeDtypeStruct((G, EO), f32),
        ],
    )(h1p, e, We1e, be1.reshape(1, H), We2, be2.reshape(1, EO), idx3)


_NPAD = 10240
_SEG = _NPAD // NC
_TRASH = 128
_ACC = _SEG + _TRASH
_KS = 80
_EPT = E // NS


def _scatter_agg(ea, dst):
    zz = jnp.zeros((_ACC // NS, EO), f32)
    mesh = plsc.VectorSubcoreMesh(core_axis_name="c", subcore_axis_name="s")

    @functools.partial(
        pl.kernel,
        mesh=mesh,
        out_type=jax.ShapeDtypeStruct((_NPAD, EO), f32),
        scratch_types=[
            pltpu.VMEM((_KS,), i32),
            pltpu.VMEM((_KS,), i32),
            pltpu.VMEM((_KS, EO), f32),
            pltpu.VMEM_SHARED((_ACC, EO), f32),
        ],
    )
    def k(ea_hbm, dst_hbm, zz_hbm, out_hbm, di, ti, rows, acc):
        c = lax.axis_index("c")
        s = lax.axis_index("s")
        zrows = _ACC // NS
        pltpu.sync_copy(zz_hbm, acc.at[pl.ds(s * zrows, zrows)])
        plsc.subcore_barrier()
        lo = c * _SEG
        base = s * _EPT

        def chunk(j, carry):
            off = base + j * _KS
            pltpu.sync_copy(dst_hbm.at[pl.ds(off, _KS)], di)
            pltpu.sync_copy(ea_hbm.at[pl.ds(off, _KS)], rows)
            for q in range(_KS // L):
                sl = pl.ds(q * L, L)
                rel = di[sl] - lo
                m = (rel >= 0) & (rel < _SEG)
                tr = _SEG + (q % (_TRASH // L)) * L + lax.iota(i32, (L,))
                ti[sl] = jnp.where(m, rel, tr)
            pltpu.sync_copy(rows, acc.at[ti], add=True)
            return carry

        lax.fori_loop(0, _EPT // _KS, chunk, 0)
        plsc.subcore_barrier()
        rpt = _SEG // NS
        pltpu.sync_copy(acc.at[pl.ds(s * rpt, rpt)],
                        out_hbm.at[pl.ds(c * _SEG + s * rpt, rpt)])

    return k(ea, dst, zz)[:N]


_TN = 400


def _node_body(x_ref, ea_ref, wa_ref, wb_ref, b1_ref, w2_ref, b2_ref, idx_ref,
               g_ref, eg_ref, wgg_ref, wgn_ref, wge_ref, bg1_ref, wg2_ref,
               bg2_ref, na_ref, ng_ref, go_ref):
    i = pl.program_id(0)
    nb = pl.num_programs(0)
    h = (jnp.dot(x_ref[...], wa_ref[...], preferred_element_type=f32)
         + jnp.dot(ea_ref[...], wb_ref[...], preferred_element_type=f32)
         + b1_ref[...])
    a = jnp.maximum(h, 0.0)
    na = jnp.dot(a, w2_ref[...], preferred_element_type=f32) + b2_ref[...]
    na_ref[...] = na
    gcol = lax.broadcasted_iota(i32, (G, _TN), 0)
    oh = (gcol == idx_ref[0]).astype(f32)
    part = jnp.dot(oh, na, preferred_element_type=f32)

    @pl.when(i == 0)
    def _():
        ng_ref[...] = part

    @pl.when(i != 0)
    def _():
        ng_ref[...] = ng_ref[...] + part

    @pl.when(i == nb - 1)
    def _():
        hg = (jnp.dot(g_ref[...], wgg_ref[...], preferred_element_type=f32)
              + jnp.dot(ng_ref[...], wgn_ref[...], preferred_element_type=f32)
              + jnp.dot(eg_ref[...], wge_ref[...], preferred_element_type=f32)
              + bg1_ref[...])
        ga = jnp.maximum(hg, 0.0)
        go_ref[...] = (jnp.dot(ga, wg2_ref[...], preferred_element_type=f32)
                       + bg2_ref[...])


def _node_global(x, eagg, node_idx, Wn1a, Wn1b, bn1, Wn2, bn2,
                 g, eg, Wg1g, Wg1n, Wg1e, bg1, Wg2, bg2):
    nb = N // _TN

    def full(r, c_):
        return pl.BlockSpec((r, c_), lambda i: (0, 0))

    idx3 = node_idx.reshape(nb, 1, _TN)
    na, _, go = pl.pallas_call(
        _node_body,
        grid=(nb,),
        in_specs=[
            pl.BlockSpec((_TN, DF), lambda i: (i, 0)),
            pl.BlockSpec((_TN, EO), lambda i: (i, 0)),
            full(DF, H),
            full(EO, H),
            full(1, H),
            full(H, NO),
            full(1, NO),
            pl.BlockSpec((1, 1, _TN), lambda i: (i, 0, 0)),
            full(G, DG),
            full(G, EO),
            full(DG, H),
            full(NO, H),
            full(EO, H),
            full(1, H),
            full(H, GO),
            full(1, GO),
        ],
        out_specs=[
            pl.BlockSpec((_TN, NO), lambda i: (i, 0)),
            pl.BlockSpec((G, NO), lambda i: (0, 0)),
            pl.BlockSpec((G, GO), lambda i: (0, 0)),
        ],
        out_shape=[
            jax.ShapeDtypeStruct((N, NO), f32),
            jax.ShapeDtypeStruct((G, NO), f32),
            jax.ShapeDtypeStruct((G, GO), f32),
        ],
    )(x, eagg, Wn1a, Wn1b, bn1.reshape(1, H), Wn2, bn2.reshape(1, NO), idx3,
      g, eg, Wg1g, Wg1n, Wg1e, bg1.reshape(1, H), Wg2, bg2.reshape(1, GO))
    return na, go


def kernel(x, e, edges, g, node_idx, edge_idx, We1, be1, We2, be2,
           Wn1, bn1, Wn2, bn2, Wg1, bg1, Wg2, bg2):
    src = edges[0]
    dst = edges[1]
    We1e = We1[:DE]
    We1s = We1[DE:DE + DF]
    We1d = We1[DE + DF:]
    Wn1a = Wn1[:DF]
    Wn1b = Wn1[DF:]
    Wg1g = Wg1[:DG]
    Wg1n = Wg1[DG:DG + NO]
    Wg1e = Wg1[DG + NO:]

    U, V = _compute_uv(x, We1s, We1d)
    h1p = _gather_h1p(U, V, src, dst)
    edge_attr, edge_g = _edge_block(h1p, e, We1e, be1, We2, be2, edge_idx)
    edge_agg = _scatter_agg(edge_attr, dst)
    node_attr, global_attr = _node_global(
        x, edge_agg, node_idx, Wn1a, Wn1b, bn1, Wn2, bn2,
        g, edge_g, Wg1g, Wg1n, Wg1e, bg1, Wg2, bg2)
    return (edge_attr, node_attr, global_attr)


# speedup vs baseline: 1.4739x; 1.4739x over previous
import functools

import jax
import jax.numpy as jnp
from jax import lax
from jax.experimental import pallas as pl
from jax.experimental.pallas import tpu as pltpu
from jax.experimental.pallas import tpu_sc as plsc

N = 10000
E = 160000
DF = 256
DE = 16
DG = 128
G = 16
H = 512
EO = 256
NO = 256
GO = 128

f32 = jnp.float32
i32 = jnp.int32

NC = 2
NS = 16
L = 16
NW = NC * NS


_UV_TILE = 2000


def _uv_body(x_ref, ws_ref, wd_ref, u_ref, v_ref):
    xv = x_ref[...]
    u_ref[...] = jnp.dot(xv, ws_ref[...], preferred_element_type=f32)
    v_ref[...] = jnp.dot(xv, wd_ref[...], preferred_element_type=f32)


def _compute_uv(x, Ws, Wd):
    nb = N // _UV_TILE
    return pl.pallas_call(
        _uv_body,
        grid=(nb,),
        in_specs=[
            pl.BlockSpec((_UV_TILE, DF), lambda i: (i, 0)),
            pl.BlockSpec((DF, H), lambda i: (0, 0)),
            pl.BlockSpec((DF, H), lambda i: (0, 0)),
        ],
        out_specs=[
            pl.BlockSpec((_UV_TILE, H), lambda i: (i, 0)),
            pl.BlockSpec((_UV_TILE, H), lambda i: (i, 0)),
        ],
        out_shape=[
            jax.ShapeDtypeStruct((N, H), f32),
            jax.ShapeDtypeStruct((N, H), f32),
        ],
    )(x, Ws, Wd)


_KG = 40
_EPW = E // NW


def _gather_h1p(U, V, src, dst):
    mesh = plsc.VectorSubcoreMesh(core_axis_name="c", subcore_axis_name="s")

    @functools.partial(
        pl.kernel,
        mesh=mesh,
        out_type=jax.ShapeDtypeStruct((E, H), f32),
        scratch_types=[
            pltpu.VMEM((_EPW,), i32),
            pltpu.VMEM((_EPW,), i32),
            pltpu.VMEM((_KG, H), f32),
            pltpu.VMEM((_KG, H), f32),
            pltpu.SemaphoreType.DMA,
            pltpu.SemaphoreType.DMA,
        ],
    )
    def k(u_hbm, v_hbm, src_hbm, dst_hbm, out_hbm, si, di, bu, bv, sem_u, sem_v):
        wid = lax.axis_index("s") * NC + lax.axis_index("c")
        base = wid * _EPW
        pltpu.sync_copy(src_hbm.at[pl.ds(base, _EPW)], si)
        pltpu.sync_copy(dst_hbm.at[pl.ds(base, _EPW)], di)

        def chunk(j, carry):
            off = j * _KG
            cu = pltpu.async_copy(u_hbm.at[si.at[pl.ds(off, _KG)]], bu, sem_u)
            cv = pltpu.async_copy(v_hbm.at[di.at[pl.ds(off, _KG)]], bv, sem_v)
            cu.wait()
            cv.wait()

            def addrow(r, c2):
                for q in range(H // L):
                    sl = pl.ds(q * L, L)
                    bu[r, sl] = bu[r, sl] + bv[r, sl]
                return c2

            lax.fori_loop(0, _KG, addrow, 0)
            pltpu.sync_copy(bu, out_hbm.at[pl.ds(base + off, _KG)])
            return carry

        lax.fori_loop(0, _EPW // _KG, chunk, 0)

    return k(U, V, src, dst)


_TE = 640


def _edge_body(h_ref, e_ref, w1_ref, b1_ref, w2_ref, b2_ref, idx_ref,
               ea_ref, eg_ref):
    i = pl.program_id(0)
    h1 = (h_ref[...]
          + jnp.dot(e_ref[...], w1_ref[...], preferred_element_type=f32)
          + b1_ref[...])
    a = jnp.maximum(h1, 0.0)
    ea = jnp.dot(a, w2_ref[...], preferred_element_type=f32) + b2_ref[...]
    ea_ref[...] = ea
    gcol = lax.broadcasted_iota(i32, (G, _TE), 0)
    oh = (gcol == idx_ref[0]).astype(f32)
    part = jnp.dot(oh, ea, preferred_element_type=f32)

    @pl.when(i == 0)
    def _():
        eg_ref[...] = part

    @pl.when(i != 0)
    def _():
        eg_ref[...] = eg_ref[...] + part


def _edge_block(h1p, e, We1e, be1, We2, be2, edge_idx):
    nb = E // _TE
    idx3 = edge_idx.reshape(nb, 1, _TE)
    return pl.pallas_call(
        _edge_body,
        grid=(nb,),
        in_specs=[
            pl.BlockSpec((_TE, H), lambda i: (i, 0)),
            pl.BlockSpec((_TE, DE), lambda i: (i, 0)),
            pl.BlockSpec((DE, H), lambda i: (0, 0)),
            pl.BlockSpec((1, H), lambda i: (0, 0)),
            pl.BlockSpec((H, EO), lambda i: (0, 0)),
            pl.BlockSpec((1, EO), lambda i: (0, 0)),
            pl.BlockSpec((1, 1, _TE), lambda i: (i, 0, 0)),
        ],
        out_specs=[
            pl.BlockSpec((_TE, EO), lambda i: (i, 0)),
            pl.BlockSpec((G, EO), lambda i: (0, 0)),
        ],
        out_shape=[
            jax.ShapeDtypeStruct((E, EO), f32),
            jax.ShapeDtypeStruct((G, EO), f32),
        ],
    )(h1p, e, We1e, be1.reshape(1, H), We2, be2.reshape(1, EO), idx3)


_NPAD = 10240
_ROWS_PT = _NPAD // NW
_SW = 2000
_LCAP = _SW + L


def _scatter_agg(ea, dst):
    zz = jnp.zeros((_ROWS_PT, EO), f32)
    mesh = plsc.VectorSubcoreMesh(core_axis_name="c", subcore_axis_name="s")

    @functools.partial(
        pl.kernel,
        mesh=mesh,
        compiler_params=pltpu.CompilerParams(needs_layout_passes=False),
        out_type=jax.ShapeDtypeStruct((_NPAD, EO), f32),
        scratch_types=[
            pltpu.VMEM((_SW,), i32),
            pltpu.VMEM((_LCAP,), i32),
            pltpu.VMEM((_LCAP,), i32),
            pltpu.VMEM((L, EO), f32),
            pltpu.VMEM((_ROWS_PT + 8, EO), f32),
            pltpu.SemaphoreType.DMA,
        ],
    )
    def k(ea_hbm, dst_hbm, zz_hbm, out_hbm, di, eids, rels, gbuf, acc, sem):
        w = lax.axis_index("s") * NC + lax.axis_index("c")
        lo = w * _ROWS_PT
        pltpu.sync_copy(zz_hbm, acc.at[pl.ds(0, _ROWS_PT)])

        def window(t, carry):
            wbase = t * _SW
            pltpu.sync_copy(dst_hbm.at[pl.ds(wbase, _SW)], di)

            def scan(q, n):
                v = di[pl.ds(q * L, L)]
                rel = v - lo
                m = (rel >= 0) & (rel < _ROWS_PT)
                eid = wbase + q * L + lax.iota(i32, L)
                plsc.store_compressed(eids.at[pl.ds(n, L)], eid, mask=m)
                plsc.store_compressed(rels.at[pl.ds(n, L)], rel, mask=m)
                return n + plsc.all_reduce_population_count(m)[0]

            n = lax.fori_loop(0, _SW // L, scan, 0)
            eids[pl.ds(n, L)] = wbase + lax.iota(i32, L)
            rels[pl.ds(n, L)] = jnp.full((L,), _ROWS_PT, i32)

            def flush(kk, c2):
                off = kk * L
                pltpu.async_copy(
                    ea_hbm.at[eids.at[pl.ds(off, L)]], gbuf, sem).wait()
                rel16 = rels[pl.ds(off, L)]
                for r in range(L):
                    rr = rel16[r]
                    for q in range(EO // L):
                        sl = pl.ds(q * L, L)
                        plsc.addupdate(acc.at[rr, sl], gbuf[r, sl])
                return c2

            lax.fori_loop(0, (n + L - 1) // L, flush, 0)
            return carry

        lax.fori_loop(0, E // _SW, window, 0)
        pltpu.sync_copy(acc.at[pl.ds(0, _ROWS_PT)],
                        out_hbm.at[pl.ds(lo, _ROWS_PT)])

    return k(ea, dst, zz)[:N]


_TN = 400


def _node_body(x_ref, ea_ref, wa_ref, wb_ref, b1_ref, w2_ref, b2_ref, idx_ref,
               g_ref, eg_ref, wgg_ref, wgn_ref, wge_ref, bg1_ref, wg2_ref,
               bg2_ref, na_ref, ng_ref, go_ref):
    i = pl.program_id(0)
    nb = pl.num_programs(0)
    h = (jnp.dot(x_ref[...], wa_ref[...], preferred_element_type=f32)
         + jnp.dot(ea_ref[...], wb_ref[...], preferred_element_type=f32)
         + b1_ref[...])
    a = jnp.maximum(h, 0.0)
    na = jnp.dot(a, w2_ref[...], preferred_element_type=f32) + b2_ref[...]
    na_ref[...] = na
    gcol = lax.broadcasted_iota(i32, (G, _TN), 0)
    oh = (gcol == idx_ref[0]).astype(f32)
    part = jnp.dot(oh, na, preferred_element_type=f32)

    @pl.when(i == 0)
    def _():
        ng_ref[...] = part

    @pl.when(i != 0)
    def _():
        ng_ref[...] = ng_ref[...] + part

    @pl.when(i == nb - 1)
    def _():
        hg = (jnp.dot(g_ref[...], wgg_ref[...], preferred_element_type=f32)
              + jnp.dot(ng_ref[...], wgn_ref[...], preferred_element_type=f32)
              + jnp.dot(eg_ref[...], wge_ref[...], preferred_element_type=f32)
              + bg1_ref[...])
        ga = jnp.maximum(hg, 0.0)
        go_ref[...] = (jnp.dot(ga, wg2_ref[...], preferred_element_type=f32)
                       + bg2_ref[...])


def _node_global(x, eagg, node_idx, Wn1a, Wn1b, bn1, Wn2, bn2,
                 g, eg, Wg1g, Wg1n, Wg1e, bg1, Wg2, bg2):
    nb = N // _TN

    def full(r, c_):
        return pl.BlockSpec((r, c_), lambda i: (0, 0))

    idx3 = node_idx.reshape(nb, 1, _TN)
    na, _, go = pl.pallas_call(
        _node_body,
        grid=(nb,),
        in_specs=[
            pl.BlockSpec((_TN, DF), lambda i: (i, 0)),
            pl.BlockSpec((_TN, EO), lambda i: (i, 0)),
            full(DF, H),
            full(EO, H),
            full(1, H),
            full(H, NO),
            full(1, NO),
            pl.BlockSpec((1, 1, _TN), lambda i: (i, 0, 0)),
            full(G, DG),
            full(G, EO),
            full(DG, H),
            full(NO, H),
            full(EO, H),
            full(1, H),
            full(H, GO),
            full(1, GO),
        ],
        out_specs=[
            pl.BlockSpec((_TN, NO), lambda i: (i, 0)),
            pl.BlockSpec((G, NO), lambda i: (0, 0)),
            pl.BlockSpec((G, GO), lambda i: (0, 0)),
        ],
        out_shape=[
            jax.ShapeDtypeStruct((N, NO), f32),
            jax.ShapeDtypeStruct((G, NO), f32),
            jax.ShapeDtypeStruct((G, GO), f32),
        ],
    )(x, eagg, Wn1a, Wn1b, bn1.reshape(1, H), Wn2, bn2.reshape(1, NO), idx3,
      g, eg, Wg1g, Wg1n, Wg1e, bg1.reshape(1, H), Wg2, bg2.reshape(1, GO))
    return na, go


def kernel(x, e, edges, g, node_idx, edge_idx, We1, be1, We2, be2,
           Wn1, bn1, Wn2, bn2, Wg1, bg1, Wg2, bg2):
    src = edges[0]
    dst = edges[1]
    We1e = We1[:DE]
    We1s = We1[DE:DE + DF]
    We1d = We1[DE + DF:]
    Wn1a = Wn1[:DF]
    Wn1b = Wn1[DF:]
    Wg1g = Wg1[:DG]
    Wg1n = Wg1[DG:DG + NO]
    Wg1e = Wg1[DG + NO:]

    U, V = _compute_uv(x, We1s, We1d)
    h1p = _gather_h1p(U, V, src, dst)
    edge_attr, edge_g = _edge_block(h1p, e, We1e, be1, We2, be2, edge_idx)
    edge_agg = _scatter_agg(edge_attr, dst)
    node_attr, global_attr = _node_global(
        x, edge_agg, node_idx, Wn1a, Wn1b, bn1, Wn2, bn2,
        g, edge_g, Wg1g, Wg1n, Wg1e, bg1, Wg2, bg2)
    return (edge_attr, node_attr, global_attr)


# speedup vs baseline: 2.0483x; 1.3898x over previous
import functools

import jax
import jax.numpy as jnp
from jax import lax
from jax.experimental import pallas as pl
from jax.experimental.pallas import tpu as pltpu
from jax.experimental.pallas import tpu_sc as plsc

N = 10000
E = 160000
DF = 256
DE = 16
DG = 128
G = 16
H = 512
EO = 256
NO = 256
GO = 128

f32 = jnp.float32
i32 = jnp.int32

NC = 2
NS = 16
L = 16
NW = NC * NS


_UV_TILE = 2000


def _uv_body(x_ref, ws_ref, wd_ref, u_ref, v_ref):
    xv = x_ref[...]
    u_ref[...] = jnp.dot(xv, ws_ref[...], preferred_element_type=f32)
    v_ref[...] = jnp.dot(xv, wd_ref[...], preferred_element_type=f32)


def _compute_uv(x, Ws, Wd):
    nb = N // _UV_TILE
    return pl.pallas_call(
        _uv_body,
        grid=(nb,),
        in_specs=[
            pl.BlockSpec((_UV_TILE, DF), lambda i: (i, 0)),
            pl.BlockSpec((DF, H), lambda i: (0, 0)),
            pl.BlockSpec((DF, H), lambda i: (0, 0)),
        ],
        out_specs=[
            pl.BlockSpec((_UV_TILE, H), lambda i: (i, 0)),
            pl.BlockSpec((_UV_TILE, H), lambda i: (i, 0)),
        ],
        out_shape=[
            jax.ShapeDtypeStruct((N, H), f32),
            jax.ShapeDtypeStruct((N, H), f32),
        ],
    )(x, Ws, Wd)


_KG = 40
_EPW = E // NW
_NCH = _EPW // _KG


def _gather_h1p(U, V, src, dst):
    mesh = plsc.VectorSubcoreMesh(core_axis_name="c", subcore_axis_name="s")

    @functools.partial(
        pl.kernel,
        mesh=mesh,
        out_type=jax.ShapeDtypeStruct((E, H), f32),
        scratch_types=[
            pltpu.VMEM((_EPW,), i32),
            pltpu.VMEM((_EPW,), i32),
            pltpu.VMEM((_KG, H), f32),
            pltpu.VMEM((_KG, H), f32),
            pltpu.VMEM((_KG, H), f32),
            pltpu.VMEM((_KG, H), f32),
            pltpu.SemaphoreType.DMA,
            pltpu.SemaphoreType.DMA,
        ],
    )
    def k(u_hbm, v_hbm, src_hbm, dst_hbm, out_hbm,
          si, di, bua, bva, bub, bvb, sga, sgb):
        wid = lax.axis_index("s") * NC + lax.axis_index("c")
        base = wid * _EPW
        pltpu.sync_copy(src_hbm.at[pl.ds(base, _EPW)], si)
        pltpu.sync_copy(dst_hbm.at[pl.ds(base, _EPW)], di)

        sets = ((bua, bva, sga), (bub, bvb, sgb))

        def issue(st, c):
            bu, bv, sg = st
            off = c * _KG
            pltpu.async_copy(u_hbm.at[si.at[pl.ds(off, _KG)]], bu, sg)
            pltpu.async_copy(v_hbm.at[di.at[pl.ds(off, _KG)]], bv, sg)

        def process(st, c):
            bu, bv, sg = st
            pltpu.make_async_copy(u_hbm.at[si.at[pl.ds(0, _KG)]], bu, sg).wait()
            pltpu.make_async_copy(v_hbm.at[di.at[pl.ds(0, _KG)]], bv, sg).wait()

            def addrow(r, c2):
                for q in range(H // L):
                    sl = pl.ds(q * L, L)
                    bu[r, sl] = bu[r, sl] + bv[r, sl]
                return c2

            lax.fori_loop(0, _KG, addrow, 0)
            pltpu.sync_copy(bu, out_hbm.at[pl.ds(base + c * _KG, _KG)])

        issue(sets[0], 0)

        def pair(j, carry):
            c0 = 2 * j
            issue(sets[1], c0 + 1)
            process(sets[0], c0)
            issue(sets[0], c0 + 2)
            process(sets[1], c0 + 1)
            return carry

        lax.fori_loop(0, (_NCH - 1) // 2, pair, 0)
        process(sets[0], _NCH - 1)

    return k(U, V, src, dst)


_TE = 640


def _edge_body(h_ref, e_ref, w1_ref, b1_ref, w2_ref, b2_ref, idx_ref,
               ea_ref, eg_ref):
    i = pl.program_id(0)
    h1 = (h_ref[...].astype(f32)
          + jnp.dot(e_ref[...], w1_ref[...], preferred_element_type=f32)
          + b1_ref[...])
    a = jnp.maximum(h1, 0.0).astype(jnp.bfloat16)
    ea = jnp.dot(a, w2_ref[...], preferred_element_type=f32) + b2_ref[...]
    ea_ref[...] = ea
    gcol = lax.broadcasted_iota(i32, (G, _TE), 0)
    oh = (gcol == idx_ref[0]).astype(f32)
    part = jnp.dot(oh, ea, preferred_element_type=f32)

    @pl.when(i == 0)
    def _():
        eg_ref[...] = part

    @pl.when(i != 0)
    def _():
        eg_ref[...] = eg_ref[...] + part


def _edge_block(h1p, e, We1e, be1, We2, be2, edge_idx):
    nb = E // _TE
    idx3 = edge_idx.reshape(nb, 1, _TE)
    return pl.pallas_call(
        _edge_body,
        grid=(nb,),
        in_specs=[
            pl.BlockSpec((_TE, H), lambda i: (i, 0)),
            pl.BlockSpec((_TE, DE), lambda i: (i, 0)),
            pl.BlockSpec((DE, H), lambda i: (0, 0)),
            pl.BlockSpec((1, H), lambda i: (0, 0)),
            pl.BlockSpec((H, EO), lambda i: (0, 0)),
            pl.BlockSpec((1, EO), lambda i: (0, 0)),
            pl.BlockSpec((1, 1, _TE), lambda i: (i, 0, 0)),
        ],
        out_specs=[
            pl.BlockSpec((_TE, EO), lambda i: (i, 0)),
            pl.BlockSpec((G, EO), lambda i: (0, 0)),
        ],
        out_shape=[
            jax.ShapeDtypeStruct((E, EO), f32),
            jax.ShapeDtypeStruct((G, EO), f32),
        ],
    )(h1p, e, We1e, be1.reshape(1, H), We2.astype(jnp.bfloat16),
      be2.reshape(1, EO), idx3)


_NPAD = 10240
_ROWS_PT = _NPAD // NW
_SW = 4000
_GS = 48
_LCAP = _SW + 2 * _GS


def _scatter_agg(ea, dst):
    zz = jnp.zeros((_ROWS_PT, EO), f32)
    mesh = plsc.VectorSubcoreMesh(core_axis_name="c", subcore_axis_name="s")

    @functools.partial(
        pl.kernel,
        mesh=mesh,
        compiler_params=pltpu.CompilerParams(needs_layout_passes=False),
        out_type=jax.ShapeDtypeStruct((_NPAD, EO), f32),
        scratch_types=[
            pltpu.VMEM((_SW,), i32),
            pltpu.VMEM((_LCAP,), i32),
            pltpu.VMEM((_LCAP,), i32),
            pltpu.VMEM((_GS, EO), f32),
            pltpu.VMEM((_GS, EO), f32),
            pltpu.VMEM((_ROWS_PT + 8, EO), f32),
            pltpu.SemaphoreType.DMA,
            pltpu.SemaphoreType.DMA,
        ],
    )
    def k(ea_hbm, dst_hbm, zz_hbm, out_hbm, di, eids, rels, gba, gbb, acc,
          sma, smb):
        w = lax.axis_index("s") * NC + lax.axis_index("c")
        lo = w * _ROWS_PT
        pltpu.sync_copy(zz_hbm, acc.at[pl.ds(0, _ROWS_PT)])

        gsets = ((gba, sma), (gbb, smb))

        def issue(st, kk):
            gb, sm = st
            pltpu.async_copy(ea_hbm.at[eids.at[pl.ds(kk * _GS, _GS)]], gb, sm)

        def wait_consume(st, kk):
            gb, sm = st
            pltpu.make_async_copy(
                ea_hbm.at[eids.at[pl.ds(0, _GS)]], gb, sm).wait()

            def blk(r16, c3):
                rel16 = rels[pl.ds(kk * _GS + r16 * L, L)]
                for r in range(L):
                    rr = rel16[r]
                    for q in range(EO // L):
                        sl = pl.ds(q * L, L)
                        plsc.addupdate(acc.at[rr, sl], gb[r16 * L + r, sl])
                return c3

            lax.fori_loop(0, _GS // L, blk, 0)

        def flush(nf):
            @pl.when(nf > 0)
            def _():
                issue(gsets[0], 0)

            def step(kk, c2):
                @pl.when(lax.rem(kk, 2) == 0)
                def _():
                    @pl.when(kk + 1 < nf)
                    def _():
                        issue(gsets[1], kk + 1)
                    wait_consume(gsets[0], kk)

                @pl.when(lax.rem(kk, 2) == 1)
                def _():
                    @pl.when(kk + 1 < nf)
                    def _():
                        issue(gsets[0], kk + 1)
                    wait_consume(gsets[1], kk)

                return c2

            lax.fori_loop(0, nf, step, 0)

        def window(t, n):
            wbase = t * _SW
            pltpu.sync_copy(dst_hbm.at[pl.ds(wbase, _SW)], di)

            def scan(q, nn):
                v = di[pl.ds(q * L, L)]
                rel = v - lo
                m = (rel >= 0) & (rel < _ROWS_PT)
                eid = wbase + q * L + lax.iota(i32, L)
                plsc.store_compressed(eids.at[pl.ds(nn, L)], eid, mask=m)
                plsc.store_compressed(rels.at[pl.ds(nn, L)], rel, mask=m)
                return nn + plsc.all_reduce_population_count(m)[0]

            n = lax.fori_loop(0, _SW // L, scan, n, unroll=2)
            nf = n // _GS
            flush(nf)
            fb = nf * _GS
            for i in range(_GS // L):
                sl_dst = pl.ds(i * L, L)
                sl_src = pl.ds(fb + i * L, L)
                eids[sl_dst] = eids[sl_src]
                rels[sl_dst] = rels[sl_src]
            return n - fb

        n = lax.fori_loop(0, E // _SW, window, 0)
        for i in range(_GS // L):
            eids[pl.ds(n + i * L, L)] = i * L + lax.iota(i32, L)
            rels[pl.ds(n + i * L, L)] = jnp.full((L,), _ROWS_PT, i32)
        flush((n + _GS - 1) // _GS)
        pltpu.sync_copy(acc.at[pl.ds(0, _ROWS_PT)],
                        out_hbm.at[pl.ds(lo, _ROWS_PT)])

    return k(ea, dst, zz)[:N]


_TN = 400


def _node_body(x_ref, ea_ref, wa_ref, wb_ref, b1_ref, w2_ref, b2_ref, idx_ref,
               g_ref, eg_ref, wgg_ref, wgn_ref, wge_ref, bg1_ref, wg2_ref,
               bg2_ref, na_ref, ng_ref, go_ref):
    i = pl.program_id(0)
    nb = pl.num_programs(0)
    h = (jnp.dot(x_ref[...].astype(jnp.bfloat16), wa_ref[...],
                 preferred_element_type=f32)
         + jnp.dot(ea_ref[...].astype(jnp.bfloat16), wb_ref[...],
                   preferred_element_type=f32)
         + b1_ref[...])
    a = jnp.maximum(h, 0.0).astype(jnp.bfloat16)
    na = jnp.dot(a, w2_ref[...], preferred_element_type=f32) + b2_ref[...]
    na_ref[...] = na
    gcol = lax.broadcasted_iota(i32, (G, _TN), 0)
    oh = (gcol == idx_ref[0]).astype(f32)
    part = jnp.dot(oh, na, preferred_element_type=f32)

    @pl.when(i == 0)
    def _():
        ng_ref[...] = part

    @pl.when(i != 0)
    def _():
        ng_ref[...] = ng_ref[...] + part

    @pl.when(i == nb - 1)
    def _():
        hg = (jnp.dot(g_ref[...], wgg_ref[...], preferred_element_type=f32)
              + jnp.dot(ng_ref[...], wgn_ref[...], preferred_element_type=f32)
              + jnp.dot(eg_ref[...], wge_ref[...], preferred_element_type=f32)
              + bg1_ref[...])
        ga = jnp.maximum(hg, 0.0)
        go_ref[...] = (jnp.dot(ga, wg2_ref[...], preferred_element_type=f32)
                       + bg2_ref[...])


def _node_global(x, eagg, node_idx, Wn1a, Wn1b, bn1, Wn2, bn2,
                 g, eg, Wg1g, Wg1n, Wg1e, bg1, Wg2, bg2):
    nb = N // _TN

    def full(r, c_):
        return pl.BlockSpec((r, c_), lambda i: (0, 0))

    idx3 = node_idx.reshape(nb, 1, _TN)
    na, _, go = pl.pallas_call(
        _node_body,
        grid=(nb,),
        in_specs=[
            pl.BlockSpec((_TN, DF), lambda i: (i, 0)),
            pl.BlockSpec((_TN, EO), lambda i: (i, 0)),
            full(DF, H),
            full(EO, H),
            full(1, H),
            full(H, NO),
            full(1, NO),
            pl.BlockSpec((1, 1, _TN), lambda i: (i, 0, 0)),
            full(G, DG),
            full(G, EO),
            full(DG, H),
            full(NO, H),
            full(EO, H),
            full(1, H),
            full(H, GO),
            full(1, GO),
        ],
        out_specs=[
            pl.BlockSpec((_TN, NO), lambda i: (i, 0)),
            pl.BlockSpec((G, NO), lambda i: (0, 0)),
            pl.BlockSpec((G, GO), lambda i: (0, 0)),
        ],
        out_shape=[
            jax.ShapeDtypeStruct((N, NO), f32),
            jax.ShapeDtypeStruct((G, NO), f32),
            jax.ShapeDtypeStruct((G, GO), f32),
        ],
    )(x, eagg, Wn1a.astype(jnp.bfloat16), Wn1b.astype(jnp.bfloat16),
      bn1.reshape(1, H), Wn2.astype(jnp.bfloat16), bn2.reshape(1, NO), idx3,
      g, eg, Wg1g, Wg1n, Wg1e, bg1.reshape(1, H), Wg2, bg2.reshape(1, GO))
    return na, go


def kernel(x, e, edges, g, node_idx, edge_idx, We1, be1, We2, be2,
           Wn1, bn1, Wn2, bn2, Wg1, bg1, Wg2, bg2):
    src = edges[0]
    dst = edges[1]
    We1e = We1[:DE]
    We1s = We1[DE:DE + DF]
    We1d = We1[DE + DF:]
    Wn1a = Wn1[:DF]
    Wn1b = Wn1[DF:]
    Wg1g = Wg1[:DG]
    Wg1n = Wg1[DG:DG + NO]
    Wg1e = Wg1[DG + NO:]

    U, V = _compute_uv(x, We1s, We1d)
    h1p = _gather_h1p(U, V, src, dst)
    edge_attr, edge_g = _edge_block(h1p, e, We1e, be1, We2, be2, edge_idx)
    edge_agg = _scatter_agg(edge_attr, dst)
    node_attr, global_attr = _node_global(
        x, edge_agg, node_idx, Wn1a, Wn1b, bn1, Wn2, bn2,
        g, edge_g, Wg1g, Wg1n, Wg1e, bg1, Wg2, bg2)
    return (edge_attr, node_attr, global_attr)


# speedup vs baseline: 2.2355x; 1.0914x over previous
import functools

import jax
import jax.numpy as jnp
from jax import lax
from jax.experimental import pallas as pl
from jax.experimental.pallas import tpu as pltpu
from jax.experimental.pallas import tpu_sc as plsc

N = 10000
E = 160000
DF = 256
DE = 16
DG = 128
G = 16
H = 512
EO = 256
NO = 256
GO = 128

f32 = jnp.float32
i32 = jnp.int32

NC = 2
NS = 16
L = 16
NW = NC * NS


_UV_TILE = 2000


_HP = H // 2


def _pack_rows(u):
    lo = jax.lax.bitcast_convert_type(
        u[:, :_HP].astype(jnp.bfloat16), jnp.uint16).astype(jnp.uint32)
    hi = jax.lax.bitcast_convert_type(
        u[:, _HP:].astype(jnp.bfloat16), jnp.uint16).astype(jnp.uint32)
    return jax.lax.bitcast_convert_type((hi << 16) | lo, i32)


def _unpack_rows(p):
    u = jax.lax.bitcast_convert_type(p, jnp.uint32)
    lo = jax.lax.bitcast_convert_type(u << 16, f32)
    hi = jax.lax.bitcast_convert_type(u & jnp.uint32(0xFFFF0000), f32)
    return jnp.concatenate([lo, hi], axis=1)


def _uv_body(x_ref, ws_ref, wd_ref, u_ref, v_ref):
    xv = x_ref[...]
    u_ref[...] = _pack_rows(jnp.dot(xv, ws_ref[...], preferred_element_type=f32))
    v_ref[...] = _pack_rows(jnp.dot(xv, wd_ref[...], preferred_element_type=f32))


def _compute_uv(x, Ws, Wd):
    nb = N // _UV_TILE
    return pl.pallas_call(
        _uv_body,
        grid=(nb,),
        in_specs=[
            pl.BlockSpec((_UV_TILE, DF), lambda i: (i, 0)),
            pl.BlockSpec((DF, H), lambda i: (0, 0)),
            pl.BlockSpec((DF, H), lambda i: (0, 0)),
        ],
        out_specs=[
            pl.BlockSpec((_UV_TILE, _HP), lambda i: (i, 0)),
            pl.BlockSpec((_UV_TILE, _HP), lambda i: (i, 0)),
        ],
        out_shape=[
            jax.ShapeDtypeStruct((N, _HP), i32),
            jax.ShapeDtypeStruct((N, _HP), i32),
        ],
    )(x, Ws, Wd)


_KG = 40
_EPW = E // NW
_NCH = _EPW // _KG


def _gather_h1p(U, V, src, dst):
    mesh = plsc.VectorSubcoreMesh(core_axis_name="c", subcore_axis_name="s")

    @functools.partial(
        pl.kernel,
        mesh=mesh,
        out_type=[
            jax.ShapeDtypeStruct((E, _HP), i32),
            jax.ShapeDtypeStruct((E, _HP), i32),
        ],
        scratch_types=[
            pltpu.VMEM((_EPW,), i32),
            pltpu.VMEM((_EPW,), i32),
            pltpu.VMEM((_KG, _HP), i32),
            pltpu.VMEM((_KG, _HP), i32),
            pltpu.VMEM((_KG, _HP), i32),
            pltpu.VMEM((_KG, _HP), i32),
            pltpu.SemaphoreType.DMA,
            pltpu.SemaphoreType.DMA,
            pltpu.SemaphoreType.DMA,
            pltpu.SemaphoreType.DMA,
        ],
    )
    def k(u_hbm, v_hbm, src_hbm, dst_hbm, outs_hbm, outd_hbm,
          si, di, bua, bva, bub, bvb, sga, sgb, ssa, ssb):
        wid = lax.axis_index("s") * NC + lax.axis_index("c")
        base = wid * _EPW
        pltpu.sync_copy(src_hbm.at[pl.ds(base, _EPW)], si)
        pltpu.sync_copy(dst_hbm.at[pl.ds(base, _EPW)], di)

        sets = ((bua, bva, sga, ssa), (bub, bvb, sgb, ssb))

        def issue(st, c):
            bu, bv, sg, ss = st
            off = c * _KG

            @pl.when(c >= 2)
            def _():
                pltpu.make_async_copy(bu, outs_hbm.at[pl.ds(base, _KG)], ss).wait()
                pltpu.make_async_copy(bv, outd_hbm.at[pl.ds(base, _KG)], ss).wait()

            pltpu.async_copy(u_hbm.at[si.at[pl.ds(off, _KG)]], bu, sg)
            pltpu.async_copy(v_hbm.at[di.at[pl.ds(off, _KG)]], bv, sg)

        def process(st, c):
            bu, bv, sg, ss = st
            pltpu.make_async_copy(u_hbm.at[si.at[pl.ds(0, _KG)]], bu, sg).wait()
            pltpu.make_async_copy(v_hbm.at[di.at[pl.ds(0, _KG)]], bv, sg).wait()
            off = base + c * _KG
            pltpu.async_copy(bu, outs_hbm.at[pl.ds(off, _KG)], ss)
            pltpu.async_copy(bv, outd_hbm.at[pl.ds(off, _KG)], ss)

        issue(sets[0], 0)

        def pair(j, carry):
            c0 = 2 * j
            issue(sets[1], c0 + 1)
            process(sets[0], c0)
            issue(sets[0], c0 + 2)
            process(sets[1], c0 + 1)
            return carry

        lax.fori_loop(0, (_NCH - 1) // 2, pair, 0)
        process(sets[0], _NCH - 1)
        for st in sets:
            bu, bv, sg, ss = st
            pltpu.make_async_copy(bu, outs_hbm.at[pl.ds(base, _KG)], ss).wait()
            pltpu.make_async_copy(bv, outd_hbm.at[pl.ds(base, _KG)], ss).wait()

    return k(U, V, src, dst)


_TE = 640


def _edge_body(hs_ref, hd_ref, e_ref, w1_ref, b1_ref, w2_ref, b2_ref, idx_ref,
               ea_ref, eg_ref):
    i = pl.program_id(0)
    h1 = (_unpack_rows(hs_ref[...]) + _unpack_rows(hd_ref[...])
          + jnp.dot(e_ref[...], w1_ref[...], preferred_element_type=f32)
          + b1_ref[...])
    a = jnp.maximum(h1, 0.0).astype(jnp.bfloat16)
    ea = jnp.dot(a, w2_ref[...], preferred_element_type=f32) + b2_ref[...]
    ea_ref[...] = ea
    gcol = lax.broadcasted_iota(i32, (G, _TE), 0)
    oh = (gcol == idx_ref[0]).astype(f32)
    part = jnp.dot(oh, ea, preferred_element_type=f32)

    @pl.when(i == 0)
    def _():
        eg_ref[...] = part

    @pl.when(i != 0)
    def _():
        eg_ref[...] = eg_ref[...] + part


def _edge_block(h1s, h1d, e, We1e, be1, We2, be2, edge_idx):
    nb = E // _TE
    idx3 = edge_idx.reshape(nb, 1, _TE)
    return pl.pallas_call(
        _edge_body,
        grid=(nb,),
        in_specs=[
            pl.BlockSpec((_TE, _HP), lambda i: (i, 0)),
            pl.BlockSpec((_TE, _HP), lambda i: (i, 0)),
            pl.BlockSpec((_TE, DE), lambda i: (i, 0)),
            pl.BlockSpec((DE, H), lambda i: (0, 0)),
            pl.BlockSpec((1, H), lambda i: (0, 0)),
            pl.BlockSpec((H, EO), lambda i: (0, 0)),
            pl.BlockSpec((1, EO), lambda i: (0, 0)),
            pl.BlockSpec((1, 1, _TE), lambda i: (i, 0, 0)),
        ],
        out_specs=[
            pl.BlockSpec((_TE, EO), lambda i: (i, 0)),
            pl.BlockSpec((G, EO), lambda i: (0, 0)),
        ],
        out_shape=[
            jax.ShapeDtypeStruct((E, EO), f32),
            jax.ShapeDtypeStruct((G, EO), f32),
        ],
    )(h1s, h1d, e, We1e, be1.reshape(1, H), We2.astype(jnp.bfloat16),
      be2.reshape(1, EO), idx3)


_NPAD = 10240
_ROWS_PT = _NPAD // NW
_SW = 4000
_GS = 48
_LCAP = _SW + 2 * _GS


def _scatter_agg(ea, dst):
    zz = jnp.zeros((_ROWS_PT, EO), f32)
    mesh = plsc.VectorSubcoreMesh(core_axis_name="c", subcore_axis_name="s")

    @functools.partial(
        pl.kernel,
        mesh=mesh,
        compiler_params=pltpu.CompilerParams(needs_layout_passes=False),
        out_type=jax.ShapeDtypeStruct((_NPAD, EO), f32),
        scratch_types=[
            pltpu.VMEM((_SW,), i32),
            pltpu.VMEM((_LCAP,), i32),
            pltpu.VMEM((_LCAP,), i32),
            pltpu.VMEM((_GS, EO), f32),
            pltpu.VMEM((_GS, EO), f32),
            pltpu.VMEM((_ROWS_PT + 8, EO), f32),
            pltpu.SemaphoreType.DMA,
            pltpu.SemaphoreType.DMA,
        ],
    )
    def k(ea_hbm, dst_hbm, zz_hbm, out_hbm, di, eids, rels, gba, gbb, acc,
          sma, smb):
        w = lax.axis_index("s") * NC + lax.axis_index("c")
        lo = w * _ROWS_PT
        pltpu.sync_copy(zz_hbm, acc.at[pl.ds(0, _ROWS_PT)])

        gsets = ((gba, sma), (gbb, smb))

        def issue(st, kk):
            gb, sm = st
            pltpu.async_copy(ea_hbm.at[eids.at[pl.ds(kk * _GS, _GS)]], gb, sm)

        def wait_consume(st, kk):
            gb, sm = st
            pltpu.make_async_copy(
                ea_hbm.at[eids.at[pl.ds(0, _GS)]], gb, sm).wait()

            def blk(r16, c3):
                rel16 = rels[pl.ds(kk * _GS + r16 * L, L)]
                for r in range(L):
                    rr = rel16[r]
                    for q in range(EO // L):
                        sl = pl.ds(q * L, L)
                        plsc.addupdate(acc.at[rr, sl], gb[r16 * L + r, sl])
                return c3

            lax.fori_loop(0, _GS // L, blk, 0)

        def flush(nf):
            @pl.when(nf > 0)
            def _():
                issue(gsets[0], 0)

            def step(kk, c2):
                @pl.when(lax.rem(kk, 2) == 0)
                def _():
                    @pl.when(kk + 1 < nf)
                    def _():
                        issue(gsets[1], kk + 1)
                    wait_consume(gsets[0], kk)

                @pl.when(lax.rem(kk, 2) == 1)
                def _():
                    @pl.when(kk + 1 < nf)
                    def _():
                        issue(gsets[0], kk + 1)
                    wait_consume(gsets[1], kk)

                return c2

            lax.fori_loop(0, nf, step, 0)

        def window(t, n):
            wbase = t * _SW
            pltpu.sync_copy(dst_hbm.at[pl.ds(wbase, _SW)], di)

            def scan(q, nn):
                v = di[pl.ds(q * L, L)]
                rel = v - lo
                m = (rel >= 0) & (rel < _ROWS_PT)
                eid = wbase + q * L + lax.iota(i32, L)
                plsc.store_compressed(eids.at[pl.ds(nn, L)], eid, mask=m)
                plsc.store_compressed(rels.at[pl.ds(nn, L)], rel, mask=m)
                return nn + plsc.all_reduce_population_count(m)[0]

            n = lax.fori_loop(0, _SW // L, scan, n, unroll=2)
            nf = n // _GS
            flush(nf)
            fb = nf * _GS
            for i in range(_GS // L):
                sl_dst = pl.ds(i * L, L)
                sl_src = pl.ds(fb + i * L, L)
                eids[sl_dst] = eids[sl_src]
                rels[sl_dst] = rels[sl_src]
            return n - fb

        n = lax.fori_loop(0, E // _SW, window, 0)
        for i in range(_GS // L):
            eids[pl.ds(n + i * L, L)] = i * L + lax.iota(i32, L)
            rels[pl.ds(n + i * L, L)] = jnp.full((L,), _ROWS_PT, i32)
        flush((n + _GS - 1) // _GS)
        pltpu.sync_copy(acc.at[pl.ds(0, _ROWS_PT)],
                        out_hbm.at[pl.ds(lo, _ROWS_PT)])

    return k(ea, dst, zz)[:N]


_TN = 400


def _node_body(x_ref, ea_ref, wa_ref, wb_ref, b1_ref, w2_ref, b2_ref, idx_ref,
               g_ref, eg_ref, wgg_ref, wgn_ref, wge_ref, bg1_ref, wg2_ref,
               bg2_ref, na_ref, ng_ref, go_ref):
    i = pl.program_id(0)
    nb = pl.num_programs(0)
    h = (jnp.dot(x_ref[...].astype(jnp.bfloat16), wa_ref[...],
                 preferred_element_type=f32)
         + jnp.dot(ea_ref[...].astype(jnp.bfloat16), wb_ref[...],
                   preferred_element_type=f32)
         + b1_ref[...])
    a = jnp.maximum(h, 0.0).astype(jnp.bfloat16)
    na = jnp.dot(a, w2_ref[...], preferred_element_type=f32) + b2_ref[...]
    na_ref[...] = na
    gcol = lax.broadcasted_iota(i32, (G, _TN), 0)
    oh = (gcol == idx_ref[0]).astype(f32)
    part = jnp.dot(oh, na, preferred_element_type=f32)

    @pl.when(i == 0)
    def _():
        ng_ref[...] = part

    @pl.when(i != 0)
    def _():
        ng_ref[...] = ng_ref[...] + part

    @pl.when(i == nb - 1)
    def _():
        hg = (jnp.dot(g_ref[...], wgg_ref[...], preferred_element_type=f32)
              + jnp.dot(ng_ref[...], wgn_ref[...], preferred_element_type=f32)
              + jnp.dot(eg_ref[...], wge_ref[...], preferred_element_type=f32)
              + bg1_ref[...])
        ga = jnp.maximum(hg, 0.0)
        go_ref[...] = (jnp.dot(ga, wg2_ref[...], preferred_element_type=f32)
                       + bg2_ref[...])


def _node_global(x, eagg, node_idx, Wn1a, Wn1b, bn1, Wn2, bn2,
                 g, eg, Wg1g, Wg1n, Wg1e, bg1, Wg2, bg2):
    nb = N // _TN

    def full(r, c_):
        return pl.BlockSpec((r, c_), lambda i: (0, 0))

    idx3 = node_idx.reshape(nb, 1, _TN)
    na, _, go = pl.pallas_call(
        _node_body,
        grid=(nb,),
        in_specs=[
            pl.BlockSpec((_TN, DF), lambda i: (i, 0)),
            pl.BlockSpec((_TN, EO), lambda i: (i, 0)),
            full(DF, H),
            full(EO, H),
            full(1, H),
            full(H, NO),
            full(1, NO),
            pl.BlockSpec((1, 1, _TN), lambda i: (i, 0, 0)),
            full(G, DG),
            full(G, EO),
            full(DG, H),
            full(NO, H),
            full(EO, H),
            full(1, H),
            full(H, GO),
            full(1, GO),
        ],
        out_specs=[
            pl.BlockSpec((_TN, NO), lambda i: (i, 0)),
            pl.BlockSpec((G, NO), lambda i: (0, 0)),
            pl.BlockSpec((G, GO), lambda i: (0, 0)),
        ],
        out_shape=[
            jax.ShapeDtypeStruct((N, NO), f32),
            jax.ShapeDtypeStruct((G, NO), f32),
            jax.ShapeDtypeStruct((G, GO), f32),
        ],
    )(x, eagg, Wn1a.astype(jnp.bfloat16), Wn1b.astype(jnp.bfloat16),
      bn1.reshape(1, H), Wn2.astype(jnp.bfloat16), bn2.reshape(1, NO), idx3,
      g, eg, Wg1g, Wg1n, Wg1e, bg1.reshape(1, H), Wg2, bg2.reshape(1, GO))
    return na, go


def kernel(x, e, edges, g, node_idx, edge_idx, We1, be1, We2, be2,
           Wn1, bn1, Wn2, bn2, Wg1, bg1, Wg2, bg2):
    src = edges[0]
    dst = edges[1]
    We1e = We1[:DE]
    We1s = We1[DE:DE + DF]
    We1d = We1[DE + DF:]
    Wn1a = Wn1[:DF]
    Wn1b = Wn1[DF:]
    Wg1g = Wg1[:DG]
    Wg1n = Wg1[DG:DG + NO]
    Wg1e = Wg1[DG + NO:]

    U, V = _compute_uv(x, We1s, We1d)
    h1s, h1d = _gather_h1p(U, V, src, dst)
    edge_attr, edge_g = _edge_block(h1s, h1d, e, We1e, be1, We2, be2, edge_idx)
    edge_agg = _scatter_agg(edge_attr, dst)
    node_attr, global_attr = _node_global(
        x, edge_agg, node_idx, Wn1a, Wn1b, bn1, Wn2, bn2,
        g, edge_g, Wg1g, Wg1n, Wg1e, bg1, Wg2, bg2)
    return (edge_attr, node_attr, global_attr)


# speedup vs baseline: 2.4356x; 1.0895x over previous
import functools

import jax
import jax.numpy as jnp
from jax import lax
from jax.experimental import pallas as pl
from jax.experimental.pallas import tpu as pltpu
from jax.experimental.pallas import tpu_sc as plsc

N = 10000
E = 160000
DF = 256
DE = 16
DG = 128
G = 16
H = 512
EO = 256
NO = 256
GO = 128

f32 = jnp.float32
i32 = jnp.int32

NC = 2
NS = 16
L = 16
NW = NC * NS


_UV_TILE = 2000


_HP = H // 2


def _pack_cols(u, half):
    lo = jax.lax.bitcast_convert_type(
        u[:, :half].astype(jnp.bfloat16), jnp.uint16).astype(jnp.uint32)
    hi = jax.lax.bitcast_convert_type(
        u[:, half:].astype(jnp.bfloat16), jnp.uint16).astype(jnp.uint32)
    return jax.lax.bitcast_convert_type((hi << 16) | lo, i32)


def _pack_rows(u):
    return _pack_cols(u, _HP)


def _unpack_rows(p):
    u = jax.lax.bitcast_convert_type(p, jnp.uint32)
    lo = jax.lax.bitcast_convert_type(u << 16, f32)
    hi = jax.lax.bitcast_convert_type(u & jnp.uint32(0xFFFF0000), f32)
    return jnp.concatenate([lo, hi], axis=1)


def _uv_body(x_ref, ws_ref, wd_ref, u_ref, v_ref):
    xv = x_ref[...]
    u_ref[...] = _pack_rows(jnp.dot(xv, ws_ref[...], preferred_element_type=f32))
    v_ref[...] = _pack_rows(jnp.dot(xv, wd_ref[...], preferred_element_type=f32))


def _compute_uv(x, Ws, Wd):
    nb = N // _UV_TILE
    return pl.pallas_call(
        _uv_body,
        grid=(nb,),
        in_specs=[
            pl.BlockSpec((_UV_TILE, DF), lambda i: (i, 0)),
            pl.BlockSpec((DF, H), lambda i: (0, 0)),
            pl.BlockSpec((DF, H), lambda i: (0, 0)),
        ],
        out_specs=[
            pl.BlockSpec((_UV_TILE, _HP), lambda i: (i, 0)),
            pl.BlockSpec((_UV_TILE, _HP), lambda i: (i, 0)),
        ],
        out_shape=[
            jax.ShapeDtypeStruct((N, _HP), i32),
            jax.ShapeDtypeStruct((N, _HP), i32),
        ],
    )(x, Ws, Wd)


_KG = 40
_EPW = E // NW
_NCH = _EPW // _KG


def _gather_h1p(U, V, src, dst):
    mesh = plsc.VectorSubcoreMesh(core_axis_name="c", subcore_axis_name="s")

    @functools.partial(
        pl.kernel,
        mesh=mesh,
        out_type=[
            jax.ShapeDtypeStruct((E, _HP), i32),
            jax.ShapeDtypeStruct((E, _HP), i32),
        ],
        scratch_types=[
            pltpu.VMEM((_EPW,), i32),
            pltpu.VMEM((_EPW,), i32),
            pltpu.VMEM((_KG, _HP), i32),
            pltpu.VMEM((_KG, _HP), i32),
            pltpu.VMEM((_KG, _HP), i32),
            pltpu.VMEM((_KG, _HP), i32),
            pltpu.SemaphoreType.DMA,
            pltpu.SemaphoreType.DMA,
            pltpu.SemaphoreType.DMA,
            pltpu.SemaphoreType.DMA,
        ],
    )
    def k(u_hbm, v_hbm, src_hbm, dst_hbm, outs_hbm, outd_hbm,
          si, di, bua, bva, bub, bvb, sga, sgb, ssa, ssb):
        wid = lax.axis_index("s") * NC + lax.axis_index("c")
        base = wid * _EPW
        pltpu.sync_copy(src_hbm.at[pl.ds(base, _EPW)], si)
        pltpu.sync_copy(dst_hbm.at[pl.ds(base, _EPW)], di)

        sets = ((bua, bva, sga, ssa), (bub, bvb, sgb, ssb))

        def issue(st, c):
            bu, bv, sg, ss = st
            off = c * _KG

            @pl.when(c >= 2)
            def _():
                pltpu.make_async_copy(bu, outs_hbm.at[pl.ds(base, _KG)], ss).wait()
                pltpu.make_async_copy(bv, outd_hbm.at[pl.ds(base, _KG)], ss).wait()

            pltpu.async_copy(u_hbm.at[si.at[pl.ds(off, _KG)]], bu, sg)
            pltpu.async_copy(v_hbm.at[di.at[pl.ds(off, _KG)]], bv, sg)

        def process(st, c):
            bu, bv, sg, ss = st
            pltpu.make_async_copy(u_hbm.at[si.at[pl.ds(0, _KG)]], bu, sg).wait()
            pltpu.make_async_copy(v_hbm.at[di.at[pl.ds(0, _KG)]], bv, sg).wait()
            off = base + c * _KG
            pltpu.async_copy(bu, outs_hbm.at[pl.ds(off, _KG)], ss)
            pltpu.async_copy(bv, outd_hbm.at[pl.ds(off, _KG)], ss)

        issue(sets[0], 0)

        def pair(j, carry):
            c0 = 2 * j
            issue(sets[1], c0 + 1)
            process(sets[0], c0)
            issue(sets[0], c0 + 2)
            process(sets[1], c0 + 1)
            return carry

        lax.fori_loop(0, (_NCH - 1) // 2, pair, 0)
        process(sets[0], _NCH - 1)
        for st in sets:
            bu, bv, sg, ss = st
            pltpu.make_async_copy(bu, outs_hbm.at[pl.ds(base, _KG)], ss).wait()
            pltpu.make_async_copy(bv, outd_hbm.at[pl.ds(base, _KG)], ss).wait()

    return k(U, V, src, dst)


_TE = 640


def _edge_body(hs_ref, hd_ref, e_ref, w1_ref, b1_ref, w2_ref, b2_ref, idx_ref,
               ea_ref, eap_ref, eg_ref):
    i = pl.program_id(0)
    h1 = (_unpack_rows(hs_ref[...]) + _unpack_rows(hd_ref[...])
          + jnp.dot(e_ref[...], w1_ref[...], preferred_element_type=f32)
          + b1_ref[...])
    a = jnp.maximum(h1, 0.0).astype(jnp.bfloat16)
    ea = jnp.dot(a, w2_ref[...], preferred_element_type=f32) + b2_ref[...]
    ea_ref[...] = ea
    eap_ref[...] = _pack_cols(ea, EO // 2)
    gcol = lax.broadcasted_iota(i32, (G, _TE), 0)
    oh = (gcol == idx_ref[0]).astype(f32)
    part = jnp.dot(oh, ea, preferred_element_type=f32)

    @pl.when(i == 0)
    def _():
        eg_ref[...] = part

    @pl.when(i != 0)
    def _():
        eg_ref[...] = eg_ref[...] + part


def _edge_block(h1s, h1d, e, We1e, be1, We2, be2, edge_idx):
    nb = E // _TE
    idx3 = edge_idx.reshape(nb, 1, _TE)
    return pl.pallas_call(
        _edge_body,
        grid=(nb,),
        in_specs=[
            pl.BlockSpec((_TE, _HP), lambda i: (i, 0)),
            pl.BlockSpec((_TE, _HP), lambda i: (i, 0)),
            pl.BlockSpec((_TE, DE), lambda i: (i, 0)),
            pl.BlockSpec((DE, H), lambda i: (0, 0)),
            pl.BlockSpec((1, H), lambda i: (0, 0)),
            pl.BlockSpec((H, EO), lambda i: (0, 0)),
            pl.BlockSpec((1, EO), lambda i: (0, 0)),
            pl.BlockSpec((1, 1, _TE), lambda i: (i, 0, 0)),
        ],
        out_specs=[
            pl.BlockSpec((_TE, EO), lambda i: (i, 0)),
            pl.BlockSpec((_TE, EO // 2), lambda i: (i, 0)),
            pl.BlockSpec((G, EO), lambda i: (0, 0)),
        ],
        out_shape=[
            jax.ShapeDtypeStruct((E, EO), f32),
            jax.ShapeDtypeStruct((E, EO // 2), i32),
            jax.ShapeDtypeStruct((G, EO), f32),
        ],
    )(h1s, h1d, e, We1e, be1.reshape(1, H), We2.astype(jnp.bfloat16),
      be2.reshape(1, EO), idx3)


_NPAD = 10240
_ROWS_PT = _NPAD // NW
_SW = 4000
_GS = 64
_LCAP = _SW + 2 * _GS
_EP = EO // 2


def _scatter_agg(ea, dst):
    zz = jnp.zeros((_ROWS_PT, EO), f32)
    mesh = plsc.VectorSubcoreMesh(core_axis_name="c", subcore_axis_name="s")

    @functools.partial(
        pl.kernel,
        mesh=mesh,
        compiler_params=pltpu.CompilerParams(needs_layout_passes=False),
        out_type=jax.ShapeDtypeStruct((_NPAD, EO), f32),
        scratch_types=[
            pltpu.VMEM((_SW,), i32),
            pltpu.VMEM((_LCAP,), i32),
            pltpu.VMEM((_LCAP,), i32),
            pltpu.VMEM((_GS, _EP), i32),
            pltpu.VMEM((_GS, _EP), i32),
            pltpu.VMEM((_ROWS_PT + 8, EO), f32),
            pltpu.SemaphoreType.DMA,
            pltpu.SemaphoreType.DMA,
        ],
    )
    def k(ea_hbm, dst_hbm, zz_hbm, out_hbm, di, eids, rels, gba, gbb, acc,
          sma, smb):
        w = lax.axis_index("s") * NC + lax.axis_index("c")
        lo = w * _ROWS_PT
        pltpu.sync_copy(zz_hbm, acc.at[pl.ds(0, _ROWS_PT)])

        gsets = ((gba, sma), (gbb, smb))

        def issue(st, kk):
            gb, sm = st
            pltpu.async_copy(ea_hbm.at[eids.at[pl.ds(kk * _GS, _GS)]], gb, sm)

        def wait_consume(st, kk):
            gb, sm = st
            pltpu.make_async_copy(
                ea_hbm.at[eids.at[pl.ds(0, _GS)]], gb, sm).wait()

            def blk(r16, c3):
                rel16 = rels[pl.ds(kk * _GS + r16 * L, L)]
                for r in range(L):
                    rr = rel16[r]
                    for q in range(_EP // L):
                        u = gb[r16 * L + r, pl.ds(q * L, L)]
                        lov = plsc.bitcast(u << 16, f32)
                        hiv = plsc.bitcast(u & jnp.int32(-65536), f32)
                        plsc.addupdate(acc.at[rr, pl.ds(q * L, L)], lov)
                        plsc.addupdate(acc.at[rr, pl.ds(_EP + q * L, L)], hiv)
                return c3

            lax.fori_loop(0, _GS // L, blk, 0)

        def flush(nf):
            @pl.when(nf > 0)
            def _():
                issue(gsets[0], 0)

            def step(kk, c2):
                @pl.when(lax.rem(kk, 2) == 0)
                def _():
                    @pl.when(kk + 1 < nf)
                    def _():
                        issue(gsets[1], kk + 1)
                    wait_consume(gsets[0], kk)

                @pl.when(lax.rem(kk, 2) == 1)
                def _():
                    @pl.when(kk + 1 < nf)
                    def _():
                        issue(gsets[0], kk + 1)
                    wait_consume(gsets[1], kk)

                return c2

            lax.fori_loop(0, nf, step, 0)

        def window(t, n):
            wbase = t * _SW
            pltpu.sync_copy(dst_hbm.at[pl.ds(wbase, _SW)], di)

            def scan(q, nn):
                v = di[pl.ds(q * L, L)]
                rel = v - lo
                m = (rel >= 0) & (rel < _ROWS_PT)
                eid = wbase + q * L + lax.iota(i32, L)
                plsc.store_compressed(eids.at[pl.ds(nn, L)], eid, mask=m)
                plsc.store_compressed(rels.at[pl.ds(nn, L)], rel, mask=m)
                return nn + plsc.all_reduce_population_count(m)[0]

            n = lax.fori_loop(0, _SW // L, scan, n, unroll=2)
            nf = n // _GS
            flush(nf)
            fb = nf * _GS
            for i in range(_GS // L):
                sl_dst = pl.ds(i * L, L)
                sl_src = pl.ds(fb + i * L, L)
                eids[sl_dst] = eids[sl_src]
                rels[sl_dst] = rels[sl_src]
            return n - fb

        n = lax.fori_loop(0, E // _SW, window, 0)
        for i in range(_GS // L):
            eids[pl.ds(n + i * L, L)] = i * L + lax.iota(i32, L)
            rels[pl.ds(n + i * L, L)] = jnp.full((L,), _ROWS_PT, i32)
        flush((n + _GS - 1) // _GS)
        pltpu.sync_copy(acc.at[pl.ds(0, _ROWS_PT)],
                        out_hbm.at[pl.ds(lo, _ROWS_PT)])

    return k(ea, dst, zz)[:N]


_TN = 400


def _node_body(x_ref, ea_ref, wa_ref, wb_ref, b1_ref, w2_ref, b2_ref, idx_ref,
               g_ref, eg_ref, wgg_ref, wgn_ref, wge_ref, bg1_ref, wg2_ref,
               bg2_ref, na_ref, ng_ref, go_ref):
    i = pl.program_id(0)
    nb = pl.num_programs(0)
    h = (jnp.dot(x_ref[...].astype(jnp.bfloat16), wa_ref[...],
                 preferred_element_type=f32)
         + jnp.dot(ea_ref[...].astype(jnp.bfloat16), wb_ref[...],
                   preferred_element_type=f32)
         + b1_ref[...])
    a = jnp.maximum(h, 0.0).astype(jnp.bfloat16)
    na = jnp.dot(a, w2_ref[...], preferred_element_type=f32) + b2_ref[...]
    na_ref[...] = na
    gcol = lax.broadcasted_iota(i32, (G, _TN), 0)
    oh = (gcol == idx_ref[0]).astype(f32)
    part = jnp.dot(oh, na, preferred_element_type=f32)

    @pl.when(i == 0)
    def _():
        ng_ref[...] = part

    @pl.when(i != 0)
    def _():
        ng_ref[...] = ng_ref[...] + part

    @pl.when(i == nb - 1)
    def _():
        hg = (jnp.dot(g_ref[...], wgg_ref[...], preferred_element_type=f32)
              + jnp.dot(ng_ref[...], wgn_ref[...], preferred_element_type=f32)
              + jnp.dot(eg_ref[...], wge_ref[...], preferred_element_type=f32)
              + bg1_ref[...])
        ga = jnp.maximum(hg, 0.0)
        go_ref[...] = (jnp.dot(ga, wg2_ref[...], preferred_element_type=f32)
                       + bg2_ref[...])


def _node_global(x, eagg, node_idx, Wn1a, Wn1b, bn1, Wn2, bn2,
                 g, eg, Wg1g, Wg1n, Wg1e, bg1, Wg2, bg2):
    nb = N // _TN

    def full(r, c_):
        return pl.BlockSpec((r, c_), lambda i: (0, 0))

    idx3 = node_idx.reshape(nb, 1, _TN)
    na, _, go = pl.pallas_call(
        _node_body,
        grid=(nb,),
        in_specs=[
            pl.BlockSpec((_TN, DF), lambda i: (i, 0)),
            pl.BlockSpec((_TN, EO), lambda i: (i, 0)),
            full(DF, H),
            full(EO, H),
            full(1, H),
            full(H, NO),
            full(1, NO),
            pl.BlockSpec((1, 1, _TN), lambda i: (i, 0, 0)),
            full(G, DG),
            full(G, EO),
            full(DG, H),
            full(NO, H),
            full(EO, H),
            full(1, H),
            full(H, GO),
            full(1, GO),
        ],
        out_specs=[
            pl.BlockSpec((_TN, NO), lambda i: (i, 0)),
            pl.BlockSpec((G, NO), lambda i: (0, 0)),
            pl.BlockSpec((G, GO), lambda i: (0, 0)),
        ],
        out_shape=[
            jax.ShapeDtypeStruct((N, NO), f32),
            jax.ShapeDtypeStruct((G, NO), f32),
            jax.ShapeDtypeStruct((G, GO), f32),
        ],
    )(x, eagg, Wn1a.astype(jnp.bfloat16), Wn1b.astype(jnp.bfloat16),
      bn1.reshape(1, H), Wn2.astype(jnp.bfloat16), bn2.reshape(1, NO), idx3,
      g, eg, Wg1g, Wg1n, Wg1e, bg1.reshape(1, H), Wg2, bg2.reshape(1, GO))
    return na, go


def kernel(x, e, edges, g, node_idx, edge_idx, We1, be1, We2, be2,
           Wn1, bn1, Wn2, bn2, Wg1, bg1, Wg2, bg2):
    src = edges[0]
    dst = edges[1]
    We1e = We1[:DE]
    We1s = We1[DE:DE + DF]
    We1d = We1[DE + DF:]
    Wn1a = Wn1[:DF]
    Wn1b = Wn1[DF:]
    Wg1g = Wg1[:DG]
    Wg1n = Wg1[DG:DG + NO]
    Wg1e = Wg1[DG + NO:]

    U, V = _compute_uv(x, We1s, We1d)
    h1s, h1d = _gather_h1p(U, V, src, dst)
    edge_attr, ea_packed, edge_g = _edge_block(h1s, h1d, e, We1e, be1, We2,
                                               be2, edge_idx)
    edge_agg = _scatter_agg(ea_packed, dst)
    node_attr, global_attr = _node_global(
        x, edge_agg, node_idx, Wn1a, Wn1b, bn1, Wn2, bn2,
        g, edge_g, Wg1g, Wg1n, Wg1e, bg1, Wg2, bg2)
    return (edge_attr, node_attr, global_attr)
